# baseline (speedup 1.0000x reference)
import functools

import jax
import jax.numpy as jnp
from jax import lax
from jax.experimental import pallas as pl
from jax.experimental.pallas import tpu as pltpu
from jax.experimental.pallas import tpu_sc as plsc

N = 10000
N_PAD = 10240
E = 320000
F = 128
G = 64
BN_EPS = 1e-5

NC = 2
NS = 16
NW = NC * NS
E_PER_TILE = E // NW
CHUNK = 112
N_CHUNKS = 90
ROWS_PER_TILE = N_PAD // NS
ZROWS = 80


def _sc_segsum_body(h_hbm, ei_hbm, zero_hbm, out_hbm,
                    idx_ring, rows0, rows1, rows2, acc,
                    gs0, gs1, gs2, ss0, ss1, ss2,
                    is0, is1, is2, is3, is4, is5):
  c = lax.axis_index("c")
  s = lax.axis_index("s")
  w = c * NS + s
  chunk0 = w * N_CHUNKS
  rows = (rows0, rows1, rows2)
  gsem = (gs0, gs1, gs2)
  ssem = (ss0, ss1, ss2)
  isem = (is0, is1, is2, is3, is4, is5)

  pltpu.sync_copy(zero_hbm, rows0.at[pl.ds(0, ZROWS)])
  row0 = s * ROWS_PER_TILE
  nz = ROWS_PER_TILE // ZROWS
  for i in range(nz):
    pltpu.async_copy(rows0.at[pl.ds(0, ZROWS)],
                     acc.at[pl.ds(row0 + i * ZROWS, ZROWS)], ss0)
  for i in range(nz):
    pltpu.make_async_copy(rows0.at[pl.ds(0, ZROWS)],
                          acc.at[pl.ds(row0, ZROWS)], ss0).wait()
  plsc.subcore_barrier()

  def refill(k, sl):
    kk = lax.rem(k, N_CHUNKS)
    pltpu.async_copy(ei_hbm.at[chunk0 + kk], idx_ring.at[sl], isem[sl])

  def wait_idx(sl):
    pltpu.make_async_copy(ei_hbm.at[chunk0], idx_ring.at[sl], isem[sl]).wait()

  def gather(sl, b):
    pltpu.async_copy(h_hbm.at[idx_ring.at[sl, 0]], rows[b], gsem[b])

  def wait_gather(b):
    pltpu.make_async_copy(h_hbm.at[idx_ring.at[0, 0]], rows[b], gsem[b]).wait()

  def scatter(sl, b):
    pltpu.async_copy(rows[b], acc.at[idx_ring.at[sl, 1]], ssem[b], add=True)

  def wait_scatter(b):
    pltpu.make_async_copy(rows[b], acc.at[idx_ring.at[0, 1]], ssem[b]).wait()

  for sl in range(5):
    refill(sl, sl)
  wait_idx(0)
  gather(0, 0)
  wait_idx(1)
  gather(1, 1)

  def onechunk(k, r, first):
    b, b2 = r % 3, (r + 2) % 3
    sl2, sl5 = (r + 2) % 6, (r + 5) % 6
    wait_gather(b)
    scatter(r, b)
    if not first:
      wait_scatter(b2)
    refill(k + 5, sl5)
    wait_idx(sl2)
    gather(sl2, b2)

  for r in range(6):
    onechunk(r, r, r == 0)

  def step6(m, carry):
    k0 = 6 * m
    for r in range(6):
      onechunk(k0 + r, r, False)
    return carry

  lax.fori_loop(1, N_CHUNKS // 6, step6, 0)

  wait_scatter(2)
  wait_gather(0)
  wait_gather(1)
  wait_idx(2)
  wait_idx(3)
  wait_idx(4)
  plsc.subcore_barrier()

  nw_ = ROWS_PER_TILE // ZROWS

  def wait_rd(b):
    pltpu.make_async_copy(acc.at[pl.ds(row0, ZROWS)],
                          rows[b].at[pl.ds(0, ZROWS)], gsem[b]).wait()

  def wait_wr(b):
    pltpu.make_async_copy(rows[b].at[pl.ds(0, ZROWS)],
                          out_hbm.at[c, pl.ds(row0, ZROWS)], ssem[b]).wait()

  for i in range(nw_):
    b = i % 2
    if i >= 2:
      wait_wr(b)
    pltpu.async_copy(acc.at[pl.ds(row0 + i * ZROWS, ZROWS)],
                     rows[b].at[pl.ds(0, ZROWS)], gsem[b])
    if i >= 1:
      bp = (i - 1) % 2
      wait_rd(bp)
      pltpu.async_copy(rows[bp].at[pl.ds(0, ZROWS)],
                       out_hbm.at[c, pl.ds(row0 + (i - 1) * ZROWS, ZROWS)],
                       ssem[bp])
  bl = (nw_ - 1) % 2
  wait_rd(bl)
  pltpu.async_copy(rows[bl].at[pl.ds(0, ZROWS)],
                   out_hbm.at[c, pl.ds(row0 + (nw_ - 1) * ZROWS, ZROWS)],
                   ssem[bl])
  wait_wr((nw_ - 2) % 2)
  wait_wr(bl)


_sc_segsum = pl.kernel(
    _sc_segsum_body,
    out_type=jax.ShapeDtypeStruct((NC, N_PAD, F), jnp.float32),
    mesh=plsc.VectorSubcoreMesh(
        core_axis_name="c", subcore_axis_name="s",
        num_cores=NC, num_subcores=NS),
    scratch_types=[
        pltpu.VMEM((6, 2, CHUNK), jnp.int32),
        pltpu.VMEM((CHUNK, F), jnp.float32),
        pltpu.VMEM((CHUNK, F), jnp.float32),
        pltpu.VMEM((CHUNK, F), jnp.float32),
        pltpu.VMEM_SHARED((N_PAD, F), jnp.float32),
    ] + [pltpu.SemaphoreType.DMA] * 12,
)


ROW_BLK = 1024


def _tc_layer_body(h_ref, p_ref, w1_ref, b1_ref, w2_ref, b2_ref,
                   gamma_ref, beta_ref, out_ref):
  a = h_ref[...] + p_ref[0] + p_ref[1]
  t = jnp.maximum(jnp.dot(a, w1_ref[...],
                          preferred_element_type=jnp.float32) + b1_ref[...], 0.0)
  u = jnp.dot(t, w2_ref[...], preferred_element_type=jnp.float32) + b2_ref[...]
  scale = gamma_ref[...] * (1.0 / jnp.sqrt(1.0 + BN_EPS))
  out_ref[...] = jnp.maximum(u * scale + beta_ref[...], 0.0)


_row_spec = pl.BlockSpec((ROW_BLK, F), lambda i: (i, 0))
_par_spec = pl.BlockSpec((NC, ROW_BLK, F), lambda i: (0, i, 0))
_full = pl.BlockSpec((F, F), lambda i: (0, 0))
_vec = pl.BlockSpec((1, F), lambda i: (0, 0))


def _tc_layer(h, partials, W1, b1, W2, b2, gamma, beta):
  return pl.pallas_call(
      _tc_layer_body,
      grid=(N_PAD // ROW_BLK,),
      in_specs=[_row_spec, _par_spec, _full, _vec, _full, _vec, _vec, _vec],
      out_specs=_row_spec,
      out_shape=jax.ShapeDtypeStruct((N_PAD, F), jnp.float32),
  )(h, partials, W1, b1.reshape(1, F), W2, b2.reshape(1, F),
    gamma.reshape(1, F), beta.reshape(1, F))


def _tc_last_body(h_ref, p_ref, w1_ref, b1_ref, w2_ref, b2_ref,
                  gamma_ref, beta_ref, bidx_ref, wp_ref, bp_ref,
                  out_ref, sums_ref, cnt_ref):
  i = pl.program_id(0)

  @pl.when(i == 0)
  def _():
    sums_ref[...] = jnp.zeros((G, F), jnp.float32)
    cnt_ref[...] = jnp.zeros((G, 128), jnp.float32)

  a = h_ref[...] + p_ref[0] + p_ref[1]
  t = jnp.maximum(jnp.dot(a, w1_ref[...],
                          preferred_element_type=jnp.float32) + b1_ref[...], 0.0)
  u = jnp.dot(t, w2_ref[...], preferred_element_type=jnp.float32) + b2_ref[...]
  scale = gamma_ref[...] * (1.0 / jnp.sqrt(1.0 + BN_EPS))
  o = jnp.maximum(u * scale + beta_ref[...], 0.0)

  groups = lax.broadcasted_iota(jnp.int32, (1, G), 1)
  onehot = jnp.where(bidx_ref[...] == groups, 1.0, 0.0)
  sums_ref[...] += lax.dot_general(onehot, o, (((0,), (0,)), ((), ())),
                                   preferred_element_type=jnp.float32)
  ones = jnp.ones((ROW_BLK, 128), dtype=jnp.float32)
  cnt_ref[...] += lax.dot_general(onehot, ones, (((0,), (0,)), ((), ())),
                                  preferred_element_type=jnp.float32)

  @pl.when(i == N_PAD // ROW_BLK - 1)
  def _():
    pooled = sums_ref[...] / jnp.maximum(cnt_ref[...], 1.0)
    out = jnp.dot(pooled, wp_ref[...],
                  preferred_element_type=jnp.float32) + bp_ref[...]
    out_ref[...] = jnp.maximum(out, 0.0)


def _tc_last(h, partials, W1, b1, W2, b2, gamma, beta, bidx, Wp, bp):
  bidx_spec = pl.BlockSpec((ROW_BLK, 1), lambda i: (i, 0))
  return pl.pallas_call(
      _tc_last_body,
      grid=(N_PAD // ROW_BLK,),
      in_specs=[_row_spec, _par_spec, _full, _vec, _full, _vec, _vec, _vec,
                bidx_spec, pl.BlockSpec((F, G), lambda i: (0, 0)),
                pl.BlockSpec((1, G), lambda i: (0, 0))],
      out_specs=pl.BlockSpec((G, G), lambda i: (0, 0)),
      out_shape=jax.ShapeDtypeStruct((G, G), jnp.float32),
      scratch_shapes=[pltpu.VMEM((G, F), jnp.float32),
                      pltpu.VMEM((G, 128), jnp.float32)],
  )(h, partials, W1, b1.reshape(1, F), W2, b2.reshape(1, F),
    gamma.reshape(1, F), beta.reshape(1, F),
    bidx.reshape(N_PAD, 1), Wp, bp.reshape(1, G))


@jax.jit
def kernel(x, edge_index, batch_idx,
           W1_0, b1_0, W2_0, b2_0, gamma_0, beta_0,
           W1_1, b1_1, W2_1, b2_1, gamma_1, beta_1,
           W1_2, b1_2, W2_2, b2_2, gamma_2, beta_2,
           Wp, bp):
  src_t = edge_index[0].reshape(NW, E_PER_TILE)
  dst_t = edge_index[1].reshape(NW, E_PER_TILE)
  epad = N_CHUNKS * CHUNK - E_PER_TILE
  src_t = jnp.pad(src_t, ((0, 0), (0, epad)))
  dpad = N + (jnp.arange(epad, dtype=jnp.int32) % (N_PAD - N)).reshape(1, epad)
  dst_t = jnp.concatenate([dst_t, jnp.broadcast_to(dpad, (NW, epad))], axis=1)
  ei = jnp.stack([src_t.reshape(NW, N_CHUNKS, CHUNK),
                  dst_t.reshape(NW, N_CHUNKS, CHUNK)],
                 axis=2).reshape(NW * N_CHUNKS, 2, CHUNK)
  zero_blk = jnp.zeros((ZROWS, F), dtype=jnp.float32)
  h = jnp.pad(x, ((0, N_PAD - N), (0, 0)))
  bidx = jnp.pad(batch_idx, (0, N_PAD - N), constant_values=G)
  layers = [
      (W1_0, b1_0, W2_0, b2_0, gamma_0, beta_0),
      (W1_1, b1_1, W2_1, b2_1, gamma_1, beta_1),
  ]
  for (W1, b1, W2, b2, g, bt) in layers:
    partials = _sc_segsum(h, ei, zero_blk)
    h = _tc_layer(h, partials, W1, b1, W2, b2, g, bt)
  partials = _sc_segsum(h, ei, zero_blk)
  return _tc_last(h, partials, W1_2, b1_2, W2_2, b2_2, gamma_2, beta_2,
                  bidx, Wp, bp)


# speedup vs baseline: 1.6907x; 1.6907x over previous
import functools

import jax
import jax.numpy as jnp
from jax import lax
from jax.experimental import pallas as pl
from jax.experimental.pallas import tpu as pltpu
from jax.experimental.pallas import tpu_sc as plsc

N = 10000
N_PAD = 10240
E = 320000
F = 128
G = 64
BN_EPS = 1e-5

NC = 2
NS = 16
NW = NC * NS
E_PER_TILE = E // NW
CHUNK = 125
N_CHUNKS = E_PER_TILE // CHUNK
ROWS_PER_TILE = N_PAD // NS
ZROWS = 80


def _sc_segsum_body(h_hbm, ei_hbm, zero_hbm, out_hbm,
                    idx_ring, rows0, rows1, acc,
                    gs0, gs1, ss0, ss1, is0, is1, is2, is3):
  c = lax.axis_index("c")
  s = lax.axis_index("s")
  w = c * NS + s
  chunk0 = w * N_CHUNKS
  rows = (rows0, rows1)
  gsem = (gs0, gs1)
  ssem = (ss0, ss1)
  isem = (is0, is1, is2, is3)

  pltpu.sync_copy(zero_hbm, rows0.at[pl.ds(0, ZROWS)])
  row0 = s * ROWS_PER_TILE
  nz = ROWS_PER_TILE // ZROWS
  for i in range(nz):
    pltpu.async_copy(rows0.at[pl.ds(0, ZROWS)],
                     acc.at[pl.ds(row0 + i * ZROWS, ZROWS)], ss0)
  for i in range(nz):
    pltpu.make_async_copy(rows0.at[pl.ds(0, ZROWS)],
                          acc.at[pl.ds(row0, ZROWS)], ss0).wait()
  plsc.subcore_barrier()

  def refill(k, sl):
    kk = lax.rem(k, N_CHUNKS)
    pltpu.async_copy(ei_hbm.at[chunk0 + kk], idx_ring.at[sl], isem[sl])

  def wait_idx(sl):
    pltpu.make_async_copy(ei_hbm.at[chunk0], idx_ring.at[sl], isem[sl]).wait()

  def gather(sl, b):
    pltpu.async_copy(h_hbm.at[idx_ring.at[sl, 0]], rows[b], gsem[b])

  def wait_gather(b):
    pltpu.make_async_copy(h_hbm.at[idx_ring.at[0, 0]], rows[b], gsem[b]).wait()

  def scatter(sl, b):
    pltpu.async_copy(rows[b], acc.at[idx_ring.at[sl, 1]], ssem[b], add=True)

  def wait_scatter(b):
    pltpu.make_async_copy(rows[b], acc.at[idx_ring.at[0, 1]], ssem[b]).wait()

  refill(0, 0)
  refill(1, 1)
  wait_idx(0)
  gather(0, 0)
  wait_idx(1)
  gather(1, 1)

  def pair(k0, p):
    sl0, sl1, sl2, sl3 = (2 * p) % 4, (2 * p + 1) % 4, (2 * p + 2) % 4, (2 * p + 3) % 4
    refill(k0 + 2, sl2)
    refill(k0 + 3, sl3)
    wait_gather(0)
    scatter(sl0, 0)
    wait_scatter(0)
    wait_idx(sl2)
    gather(sl2, 0)
    wait_gather(1)
    scatter(sl1, 1)
    wait_scatter(1)
    wait_idx(sl3)
    gather(sl3, 1)

  def step4(j, carry):
    pair(4 * j, 0)
    pair(4 * j + 2, 1)
    return carry

  lax.fori_loop(0, N_CHUNKS // 4, step4, 0)

  wait_gather(0)
  wait_gather(1)
  plsc.subcore_barrier()

  nw_ = ROWS_PER_TILE // ZROWS

  def wait_rd(b):
    pltpu.make_async_copy(acc.at[pl.ds(row0, ZROWS)],
                          rows[b].at[pl.ds(0, ZROWS)], gsem[b]).wait()

  def wait_wr(b):
    pltpu.make_async_copy(rows[b].at[pl.ds(0, ZROWS)],
                          out_hbm.at[c, pl.ds(row0, ZROWS)], ssem[b]).wait()

  for i in range(nw_):
    b = i % 2
    if i >= 2:
      wait_wr(b)
    pltpu.async_copy(acc.at[pl.ds(row0 + i * ZROWS, ZROWS)],
                     rows[b].at[pl.ds(0, ZROWS)], gsem[b])
    if i >= 1:
      bp = (i - 1) % 2
      wait_rd(bp)
      pltpu.async_copy(rows[bp].at[pl.ds(0, ZROWS)],
                       out_hbm.at[c, pl.ds(row0 + (i - 1) * ZROWS, ZROWS)],
                       ssem[bp])
  bl = (nw_ - 1) % 2
  wait_rd(bl)
  pltpu.async_copy(rows[bl].at[pl.ds(0, ZROWS)],
                   out_hbm.at[c, pl.ds(row0 + (nw_ - 1) * ZROWS, ZROWS)],
                   ssem[bl])
  wait_wr((nw_ - 2) % 2)
  wait_wr(bl)


_sc_segsum = pl.kernel(
    _sc_segsum_body,
    out_type=jax.ShapeDtypeStruct((NC, N_PAD, F), jnp.float32),
    mesh=plsc.VectorSubcoreMesh(
        core_axis_name="c", subcore_axis_name="s",
        num_cores=NC, num_subcores=NS),
    scratch_types=[
        pltpu.VMEM((4, 2, CHUNK), jnp.int32),
        pltpu.VMEM((CHUNK, F), jnp.float32),
        pltpu.VMEM((CHUNK, F), jnp.float32),
        pltpu.VMEM_SHARED((N_PAD, F), jnp.float32),
    ] + [pltpu.SemaphoreType.DMA] * 8,
)


ROW_BLK = 1024


def _tc_layer_body(h_ref, p_ref, w1_ref, b1_ref, w2_ref, b2_ref,
                   gamma_ref, beta_ref, out_ref):
  a = h_ref[...] + p_ref[0] + p_ref[1]
  t = jnp.maximum(jnp.dot(a, w1_ref[...],
                          preferred_element_type=jnp.float32) + b1_ref[...], 0.0)
  u = jnp.dot(t, w2_ref[...], preferred_element_type=jnp.float32) + b2_ref[...]
  scale = gamma_ref[...] * (1.0 / jnp.sqrt(1.0 + BN_EPS))
  out_ref[...] = jnp.maximum(u * scale + beta_ref[...], 0.0)


_row_spec = pl.BlockSpec((ROW_BLK, F), lambda i: (i, 0))
_par_spec = pl.BlockSpec((NC, ROW_BLK, F), lambda i: (0, i, 0))
_full = pl.BlockSpec((F, F), lambda i: (0, 0))
_vec = pl.BlockSpec((1, F), lambda i: (0, 0))


def _tc_layer(h, partials, W1, b1, W2, b2, gamma, beta):
  return pl.pallas_call(
      _tc_layer_body,
      grid=(N_PAD // ROW_BLK,),
      in_specs=[_row_spec, _par_spec, _full, _vec, _full, _vec, _vec, _vec],
      out_specs=_row_spec,
      out_shape=jax.ShapeDtypeStruct((N_PAD, F), jnp.float32),
  )(h, partials, W1, b1.reshape(1, F), W2, b2.reshape(1, F),
    gamma.reshape(1, F), beta.reshape(1, F))


def _tc_last_body(h_ref, p_ref, w1_ref, b1_ref, w2_ref, b2_ref,
                  gamma_ref, beta_ref, bidx_ref, wp_ref, bp_ref,
                  out_ref, sums_ref, cnt_ref):
  i = pl.program_id(0)

  @pl.when(i == 0)
  def _():
    sums_ref[...] = jnp.zeros((G, F), jnp.float32)
    cnt_ref[...] = jnp.zeros((G, 128), jnp.float32)

  a = h_ref[...] + p_ref[0] + p_ref[1]
  t = jnp.maximum(jnp.dot(a, w1_ref[...],
                          preferred_element_type=jnp.float32) + b1_ref[...], 0.0)
  u = jnp.dot(t, w2_ref[...], preferred_element_type=jnp.float32) + b2_ref[...]
  scale = gamma_ref[...] * (1.0 / jnp.sqrt(1.0 + BN_EPS))
  o = jnp.maximum(u * scale + beta_ref[...], 0.0)

  groups = lax.broadcasted_iota(jnp.int32, (1, G), 1)
  onehot = jnp.where(bidx_ref[...] == groups, 1.0, 0.0)
  sums_ref[...] += lax.dot_general(onehot, o, (((0,), (0,)), ((), ())),
                                   preferred_element_type=jnp.float32)
  ones = jnp.ones((ROW_BLK, 128), dtype=jnp.float32)
  cnt_ref[...] += lax.dot_general(onehot, ones, (((0,), (0,)), ((), ())),
                                  preferred_element_type=jnp.float32)

  @pl.when(i == N_PAD // ROW_BLK - 1)
  def _():
    pooled = sums_ref[...] / jnp.maximum(cnt_ref[...], 1.0)
    out = jnp.dot(pooled, wp_ref[...],
                  preferred_element_type=jnp.float32) + bp_ref[...]
    out_ref[...] = jnp.maximum(out, 0.0)


def _tc_last(h, partials, W1, b1, W2, b2, gamma, beta, bidx, Wp, bp):
  bidx_spec = pl.BlockSpec((ROW_BLK, 1), lambda i: (i, 0))
  return pl.pallas_call(
      _tc_last_body,
      grid=(N_PAD // ROW_BLK,),
      in_specs=[_row_spec, _par_spec, _full, _vec, _full, _vec, _vec, _vec,
                bidx_spec, pl.BlockSpec((F, G), lambda i: (0, 0)),
                pl.BlockSpec((1, G), lambda i: (0, 0))],
      out_specs=pl.BlockSpec((G, G), lambda i: (0, 0)),
      out_shape=jax.ShapeDtypeStruct((G, G), jnp.float32),
      scratch_shapes=[pltpu.VMEM((G, F), jnp.float32),
                      pltpu.VMEM((G, 128), jnp.float32)],
  )(h, partials, W1, b1.reshape(1, F), W2, b2.reshape(1, F),
    gamma.reshape(1, F), beta.reshape(1, F),
    bidx.reshape(N_PAD, 1), Wp, bp.reshape(1, G))


@jax.jit
def kernel(x, edge_index, batch_idx,
           W1_0, b1_0, W2_0, b2_0, gamma_0, beta_0,
           W1_1, b1_1, W2_1, b2_1, gamma_1, beta_1,
           W1_2, b1_2, W2_2, b2_2, gamma_2, beta_2,
           Wp, bp):
  ei = edge_index.reshape(2, NW * N_CHUNKS, CHUNK).transpose(1, 0, 2)
  zero_blk = jnp.zeros((ZROWS, F), dtype=jnp.float32)
  h = jnp.pad(x, ((0, N_PAD - N), (0, 0)))
  bidx = jnp.pad(batch_idx, (0, N_PAD - N), constant_values=G)
  layers = [
      (W1_0, b1_0, W2_0, b2_0, gamma_0, beta_0),
      (W1_1, b1_1, W2_1, b2_1, gamma_1, beta_1),
  ]
  for (W1, b1, W2, b2, g, bt) in layers:
    partials = _sc_segsum(h, ei, zero_blk)
    h = _tc_layer(h, partials, W1, b1, W2, b2, g, bt)
  partials = _sc_segsum(h, ei, zero_blk)
  return _tc_last(h, partials, W1_2, b1_2, W2_2, b2_2, gamma_2, beta_2,
                  bidx, Wp, bp)
